# baseline (speedup 1.0000x reference)
import functools

import jax
import jax.numpy as jnp
from jax import lax
from jax.experimental import pallas as pl
from jax.experimental.pallas import tpu as pltpu
from jax.experimental.pallas import tpu_sc as plsc

N = 10000
D = 128
E = 320000
NC = 2
NS = 16
NW = NC * NS
EPW = E // NW
K = 80
NCHUNK = EPW // K
RPT = 624
TAIL = N - NS * RPT


def _make_sc_aggregate():
    mesh = plsc.VectorSubcoreMesh(core_axis_name="c", subcore_axis_name="s")

    @functools.partial(
        pl.kernel,
        out_type=jax.ShapeDtypeStruct((NC, N, D), jnp.float32),
        mesh=mesh,
        scratch_types=[
            pltpu.VMEM_SHARED((N, D), jnp.float32),
            pltpu.VMEM((EPW,), jnp.int32),
            pltpu.VMEM((NCHUNK, K), jnp.int32),
            pltpu.VMEM((K, D), jnp.float32),
            pltpu.VMEM((K, D), jnp.float32),
            pltpu.SemaphoreType.DMA,
            pltpu.SemaphoreType.DMA,
            pltpu.SemaphoreType.DMA,
            pltpu.SemaphoreType.DMA,
        ],
    )
    def agg(h_hbm, src_hbm, dst_hbm, out_hbm, acc, sidx, didx, r0buf, r1buf,
            gsem0, gsem1, ssem0, ssem1):
        c = lax.axis_index("c")
        s = lax.axis_index("s")
        wid = s * NC + c
        r0 = s * RPT
        pltpu.sync_copy(h_hbm.at[pl.ds(r0, RPT)], acc.at[pl.ds(r0, RPT)])

        @pl.when(s == 0)
        def _():
            pltpu.sync_copy(h_hbm.at[pl.ds(NS * RPT, TAIL)],
                            acc.at[pl.ds(NS * RPT, TAIL)])

        pltpu.sync_copy(src_hbm.at[pl.ds(wid * EPW, EPW)], sidx)
        pltpu.sync_copy(dst_hbm.at[wid], didx)
        plsc.subcore_barrier()

        def gather(g, buf, sem):
            pltpu.async_copy(h_hbm.at[sidx.at[pl.ds(g * K, K)]], buf, sem)

        def drain(buf, sem):
            pltpu.make_async_copy(h_hbm.at[pl.ds(0, K)], buf, sem).wait()

        def scatter(g, buf, sem):
            pltpu.async_copy(buf, acc.at[didx.at[g]], sem, add=True)

        gather(0, r0buf, gsem0)
        gather(1, r1buf, gsem1)

        def body(j, carry):
            a = 2 * j
            drain(r0buf, gsem0)
            scatter(a, r0buf, ssem0)
            drain(r1buf, gsem1)
            scatter(a + 1, r1buf, ssem1)
            drain(r0buf, ssem0)
            gather(a + 2, r0buf, gsem0)
            drain(r1buf, ssem1)
            gather(a + 3, r1buf, gsem1)
            return carry

        lax.fori_loop(0, (NCHUNK - 3) // 2, body, 0)
        drain(r0buf, gsem0)
        scatter(NCHUNK - 3, r0buf, ssem0)
        drain(r1buf, gsem1)
        scatter(NCHUNK - 2, r1buf, ssem1)
        drain(r0buf, ssem0)
        gather(NCHUNK - 1, r0buf, gsem0)
        drain(r1buf, ssem1)
        drain(r0buf, gsem0)
        scatter(NCHUNK - 1, r0buf, ssem0)
        drain(r0buf, ssem0)

        plsc.subcore_barrier()
        pltpu.sync_copy(acc.at[pl.ds(r0, RPT)], out_hbm.at[c, pl.ds(r0, RPT)])

        @pl.when(s == 0)
        def _():
            pltpu.sync_copy(acc.at[pl.ds(NS * RPT, TAIL)],
                            out_hbm.at[c, pl.ds(NS * RPT, TAIL)])

    return agg


_sc_aggregate = _make_sc_aggregate()


def _mlp_body(p_ref, h_ref, w1_ref, b1_ref, w2_ref, b2_ref, o_ref):
    z = p_ref[0] + p_ref[1] - h_ref[...]
    z = jnp.dot(z, w1_ref[...], preferred_element_type=jnp.float32)
    z = jnp.maximum(z + b1_ref[...], 0.0)
    z = jnp.dot(z, w2_ref[...], preferred_element_type=jnp.float32)
    o_ref[...] = jnp.maximum(z + b2_ref[...], 0.0)


_BLK = 1000


def _tc_mlp(p, h, W1, b1, W2, b2):
    grid = (N // _BLK,)
    row_spec = pl.BlockSpec((_BLK, D), lambda i: (i, 0))
    p_spec = pl.BlockSpec((NC, _BLK, D), lambda i: (0, i, 0))
    full_w = pl.BlockSpec((D, D), lambda i: (0, 0))
    full_b = pl.BlockSpec((1, D), lambda i: (0, 0))
    return pl.pallas_call(
        _mlp_body,
        grid=grid,
        in_specs=[p_spec, row_spec, full_w, full_b, full_w, full_b],
        out_specs=row_spec,
        out_shape=jax.ShapeDtypeStruct((N, D), jnp.float32),
    )(p, h, W1, b1.reshape(1, D), W2, b2.reshape(1, D))


def kernel(x, edge_index, W1_0, b1_0, W2_0, b2_0, W1_1, b1_1, W2_1, b2_1):
    src = edge_index[0]
    dst = edge_index[1].reshape(NW, NCHUNK, K)
    p = _sc_aggregate(x, src, dst)
    h1 = _tc_mlp(p, x, W1_0, b1_0, W2_0, b2_0)
    p2 = _sc_aggregate(h1, src, dst)
    return _tc_mlp(p2, h1, W1_1, b1_1, W2_1, b2_1)


# speedup vs baseline: 1.2395x; 1.2395x over previous
import functools

import jax
import jax.numpy as jnp
from jax import lax
from jax.experimental import pallas as pl
from jax.experimental.pallas import tpu as pltpu
from jax.experimental.pallas import tpu_sc as plsc

N = 10000
D = 128
E = 320000
NC = 2
NS = 16
NW = NC * NS
EPW = E // NW
K = 80
NCHUNK = EPW // K
RPT = 624
TAIL = N - NS * RPT


def _make_sc_aggregate():
    mesh = plsc.VectorSubcoreMesh(core_axis_name="c", subcore_axis_name="s")

    @functools.partial(
        pl.kernel,
        out_type=jax.ShapeDtypeStruct((NC, N, D), jnp.float32),
        mesh=mesh,
        scratch_types=[
            pltpu.VMEM_SHARED((N, D), jnp.float32),
            pltpu.VMEM((EPW,), jnp.int32),
            pltpu.VMEM((NCHUNK, K), jnp.int32),
            pltpu.VMEM((K, D), jnp.float32),
            pltpu.VMEM((K, D), jnp.float32),
            pltpu.SemaphoreType.DMA,
            pltpu.SemaphoreType.DMA,
            pltpu.SemaphoreType.DMA,
            pltpu.SemaphoreType.DMA,
        ],
    )
    def agg(h_hbm, src_hbm, dst_hbm, out_hbm, acc, sidx, didx, r0buf, r1buf,
            gsem0, gsem1, ssem0, ssem1):
        c = lax.axis_index("c")
        s = lax.axis_index("s")
        wid = s * NC + c
        r0 = s * RPT
        pltpu.sync_copy(h_hbm.at[pl.ds(r0, RPT)], acc.at[pl.ds(r0, RPT)])

        @pl.when(s == 0)
        def _():
            pltpu.sync_copy(h_hbm.at[pl.ds(NS * RPT, TAIL)],
                            acc.at[pl.ds(NS * RPT, TAIL)])

        pltpu.sync_copy(src_hbm.at[pl.ds(wid * EPW, EPW)], sidx)
        pltpu.sync_copy(dst_hbm.at[wid], didx)
        plsc.subcore_barrier()

        def gather(g, buf, sem):
            pltpu.async_copy(h_hbm.at[sidx.at[pl.ds(g * K, K)]], buf, sem)

        def drain(buf, sem):
            pltpu.make_async_copy(h_hbm.at[pl.ds(0, K)], buf, sem).wait()

        def scatter(g, buf):
            pltpu.sync_copy(buf, acc.at[didx.at[g]], add=True)

        gather(0, r0buf, gsem0)

        def body(j, carry):
            a = 2 * j
            gather(a + 1, r1buf, gsem1)
            drain(r0buf, gsem0)
            scatter(a, r0buf)
            gather(a + 2, r0buf, gsem0)
            drain(r1buf, gsem1)
            scatter(a + 1, r1buf)
            return carry

        lax.fori_loop(0, (NCHUNK - 1) // 2, body, 0)
        drain(r0buf, gsem0)
        scatter(NCHUNK - 1, r0buf)

        plsc.subcore_barrier()
        pltpu.sync_copy(acc.at[pl.ds(r0, RPT)], out_hbm.at[c, pl.ds(r0, RPT)])

        @pl.when(s == 0)
        def _():
            pltpu.sync_copy(acc.at[pl.ds(NS * RPT, TAIL)],
                            out_hbm.at[c, pl.ds(NS * RPT, TAIL)])

    return agg


_sc_aggregate = _make_sc_aggregate()


def _mlp_body(p_ref, h_ref, w1_ref, b1_ref, w2_ref, b2_ref, o_ref):
    z = p_ref[0] + p_ref[1] - h_ref[...]
    z = jnp.dot(z, w1_ref[...], preferred_element_type=jnp.float32)
    z = jnp.maximum(z + b1_ref[...], 0.0)
    z = jnp.dot(z, w2_ref[...], preferred_element_type=jnp.float32)
    o_ref[...] = jnp.maximum(z + b2_ref[...], 0.0)


_BLK = 1000


def _tc_mlp(p, h, W1, b1, W2, b2):
    grid = (N // _BLK,)
    row_spec = pl.BlockSpec((_BLK, D), lambda i: (i, 0))
    p_spec = pl.BlockSpec((NC, _BLK, D), lambda i: (0, i, 0))
    full_w = pl.BlockSpec((D, D), lambda i: (0, 0))
    full_b = pl.BlockSpec((1, D), lambda i: (0, 0))
    return pl.pallas_call(
        _mlp_body,
        grid=grid,
        in_specs=[p_spec, row_spec, full_w, full_b, full_w, full_b],
        out_specs=row_spec,
        out_shape=jax.ShapeDtypeStruct((N, D), jnp.float32),
    )(p, h, W1, b1.reshape(1, D), W2, b2.reshape(1, D))


def kernel(x, edge_index, W1_0, b1_0, W2_0, b2_0, W1_1, b1_1, W2_1, b2_1):
    src = edge_index[0]
    dst = edge_index[1].reshape(NW, NCHUNK, K)
    p = _sc_aggregate(x, src, dst)
    h1 = _tc_mlp(p, x, W1_0, b1_0, W2_0, b2_0)
    p2 = _sc_aggregate(h1, src, dst)
    return _tc_mlp(p2, h1, W1_1, b1_1, W2_1, b2_1)


# speedup vs baseline: 1.2533x; 1.0111x over previous
import functools

import jax
import jax.numpy as jnp
from jax import lax
from jax.experimental import pallas as pl
from jax.experimental.pallas import tpu as pltpu
from jax.experimental.pallas import tpu_sc as plsc

N = 10000
D = 128
E = 320000
NC = 2
NS = 16
NW = NC * NS
EPW = E // NW
K = 80
NCHUNK = EPW // K
RPT = 624
TAIL = N - NS * RPT


def _make_sc_aggregate():
    mesh = plsc.VectorSubcoreMesh(core_axis_name="c", subcore_axis_name="s")

    @functools.partial(
        pl.kernel,
        out_type=jax.ShapeDtypeStruct((NC, N, D), jnp.float32),
        mesh=mesh,
        scratch_types=[
            pltpu.VMEM_SHARED((N, D), jnp.float32),
            pltpu.VMEM((EPW,), jnp.int32),
            pltpu.VMEM((NCHUNK, K), jnp.int32),
            pltpu.VMEM((K, D), jnp.float32),
            pltpu.VMEM((K, D), jnp.float32),
            pltpu.SemaphoreType.DMA,
            pltpu.SemaphoreType.DMA,
        ],
    )
    def agg(h_hbm, src_hbm, dst_hbm, out_hbm, acc, sidx, didx,
            r0buf, r1buf, gsem0, gsem1):
        c = lax.axis_index("c")
        s = lax.axis_index("s")
        wid = s * NC + c
        pltpu.sync_copy(src_hbm.at[pl.ds(wid * EPW, EPW)], sidx)
        pltpu.sync_copy(dst_hbm.at[wid], didx)
        pltpu.async_copy(h_hbm.at[sidx.at[pl.ds(0, K)]], r0buf, gsem0)
        r0 = s * RPT
        pltpu.sync_copy(h_hbm.at[pl.ds(r0, RPT)], acc.at[pl.ds(r0, RPT)])

        @pl.when(s == 0)
        def _():
            pltpu.sync_copy(h_hbm.at[pl.ds(NS * RPT, TAIL)],
                            acc.at[pl.ds(NS * RPT, TAIL)])

        plsc.subcore_barrier()

        def gather(g, buf, sem):
            pltpu.async_copy(h_hbm.at[sidx.at[pl.ds(g * K, K)]], buf, sem)

        def drain(buf, sem):
            pltpu.make_async_copy(h_hbm.at[pl.ds(0, K)], buf, sem).wait()

        def scatter(g, buf):
            pltpu.sync_copy(buf, acc.at[didx.at[g]], add=True)

        def body(j, carry):
            a = 2 * j
            gather(a + 1, r1buf, gsem1)
            drain(r0buf, gsem0)
            scatter(a, r0buf)
            gather(a + 2, r0buf, gsem0)
            drain(r1buf, gsem1)
            scatter(a + 1, r1buf)
            return carry

        lax.fori_loop(0, (NCHUNK - 1) // 2, body, 0)
        drain(r0buf, gsem0)
        scatter(NCHUNK - 1, r0buf)

        plsc.subcore_barrier()
        pltpu.sync_copy(acc.at[pl.ds(r0, RPT)], out_hbm.at[c, pl.ds(r0, RPT)])

        @pl.when(s == 0)
        def _():
            pltpu.sync_copy(acc.at[pl.ds(NS * RPT, TAIL)],
                            out_hbm.at[c, pl.ds(NS * RPT, TAIL)])

    return agg


_sc_aggregate = _make_sc_aggregate()


def _mlp_body(p_ref, h_ref, w1_ref, b1_ref, w2_ref, b2_ref, o_ref):
    z = p_ref[0] + p_ref[1] - h_ref[...]
    z = jnp.dot(z, w1_ref[...], preferred_element_type=jnp.float32)
    z = jnp.maximum(z + b1_ref[...], 0.0)
    z = jnp.dot(z, w2_ref[...], preferred_element_type=jnp.float32)
    o_ref[...] = jnp.maximum(z + b2_ref[...], 0.0)


_BLK = 1000


def _tc_mlp(p, h, W1, b1, W2, b2):
    grid = (N // _BLK,)
    row_spec = pl.BlockSpec((_BLK, D), lambda i: (i, 0))
    p_spec = pl.BlockSpec((NC, _BLK, D), lambda i: (0, i, 0))
    full_w = pl.BlockSpec((D, D), lambda i: (0, 0))
    full_b = pl.BlockSpec((1, D), lambda i: (0, 0))
    return pl.pallas_call(
        _mlp_body,
        grid=grid,
        in_specs=[p_spec, row_spec, full_w, full_b, full_w, full_b],
        out_specs=row_spec,
        out_shape=jax.ShapeDtypeStruct((N, D), jnp.float32),
    )(p, h, W1, b1.reshape(1, D), W2, b2.reshape(1, D))


def kernel(x, edge_index, W1_0, b1_0, W2_0, b2_0, W1_1, b1_1, W2_1, b2_1):
    src = edge_index[0]
    dst = edge_index[1].reshape(NW, NCHUNK, K)
    p = _sc_aggregate(x, src, dst)
    h1 = _tc_mlp(p, x, W1_0, b1_0, W2_0, b2_0)
    p2 = _sc_aggregate(h1, src, dst)
    return _tc_mlp(p2, h1, W1_1, b1_1, W2_1, b2_1)


# speedup vs baseline: 1.2764x; 1.0185x over previous
import functools

import jax
import jax.numpy as jnp
from jax import lax
from jax.experimental import pallas as pl
from jax.experimental.pallas import tpu as pltpu
from jax.experimental.pallas import tpu_sc as plsc

N = 10000
D = 128
E = 320000
NC = 2
NS = 16
NW = NC * NS
EPW = E // NW
K = 80
NCHUNK = EPW // K
RPT = 624
TAIL = N - NS * RPT


def _make_sc_aggregate():
    mesh = plsc.VectorSubcoreMesh(core_axis_name="c", subcore_axis_name="s")

    @functools.partial(
        pl.kernel,
        out_type=jax.ShapeDtypeStruct((NC, N, D), jnp.float32),
        mesh=mesh,
        scratch_types=[
            pltpu.VMEM_SHARED((N, D), jnp.float32),
            pltpu.VMEM((EPW,), jnp.int32),
            pltpu.VMEM((NCHUNK, K), jnp.int32),
            pltpu.VMEM((K, D), jnp.float32),
            pltpu.VMEM((K, D), jnp.float32),
            pltpu.SemaphoreType.DMA,
            pltpu.SemaphoreType.DMA,
        ],
    )
    def agg(h_hbm, src_hbm, dst_hbm, out_hbm, acc, sidx, didx,
            r0buf, r1buf, gsem0, gsem1):
        c = lax.axis_index("c")
        s = lax.axis_index("s")
        wid = s * NC + c
        pltpu.sync_copy(src_hbm.at[pl.ds(wid * EPW, EPW)], sidx)
        pltpu.sync_copy(dst_hbm.at[wid], didx)
        pltpu.async_copy(h_hbm.at[sidx.at[pl.ds(0, K)]], r0buf, gsem0)
        r0 = s * RPT
        pltpu.sync_copy(h_hbm.at[pl.ds(r0, RPT)], acc.at[pl.ds(r0, RPT)])

        @pl.when(s == 0)
        def _():
            pltpu.sync_copy(h_hbm.at[pl.ds(NS * RPT, TAIL)],
                            acc.at[pl.ds(NS * RPT, TAIL)])

        plsc.subcore_barrier()

        def gather(g, buf, sem):
            pltpu.async_copy(h_hbm.at[sidx.at[pl.ds(g * K, K)]], buf, sem)

        def drain(buf, sem):
            pltpu.make_async_copy(h_hbm.at[pl.ds(0, K)], buf, sem).wait()

        def scatter(g, buf):
            pltpu.sync_copy(buf, acc.at[didx.at[g]], add=True)

        def body(j, carry):
            a = 2 * j
            gather(a + 1, r1buf, gsem1)
            drain(r0buf, gsem0)
            scatter(a, r0buf)
            gather(a + 2, r0buf, gsem0)
            drain(r1buf, gsem1)
            scatter(a + 1, r1buf)
            return carry

        lax.fori_loop(0, (NCHUNK - 1) // 2, body, 0)
        drain(r0buf, gsem0)
        scatter(NCHUNK - 1, r0buf)

        plsc.subcore_barrier()
        pltpu.sync_copy(acc.at[pl.ds(r0, RPT)], out_hbm.at[c, pl.ds(r0, RPT)])

        @pl.when(s == 0)
        def _():
            pltpu.sync_copy(acc.at[pl.ds(NS * RPT, TAIL)],
                            out_hbm.at[c, pl.ds(NS * RPT, TAIL)])

    return agg


_sc_aggregate = _make_sc_aggregate()


def _mlp_body(p_ref, h_ref, w1_ref, b1_ref, w2_ref, b2_ref, o_ref):
    z = p_ref[0] + p_ref[1] - h_ref[...]
    z = jnp.dot(z, w1_ref[...], preferred_element_type=jnp.float32)
    z = jnp.maximum(z + b1_ref[...], 0.0)
    z = jnp.dot(z, w2_ref[...], preferred_element_type=jnp.float32)
    o_ref[...] = jnp.maximum(z + b2_ref[...], 0.0)


_BLK = 2000


def _tc_mlp(p, h, W1, b1, W2, b2):
    grid = (N // _BLK,)
    row_spec = pl.BlockSpec((_BLK, D), lambda i: (i, 0))
    p_spec = pl.BlockSpec((NC, _BLK, D), lambda i: (0, i, 0))
    full_w = pl.BlockSpec((D, D), lambda i: (0, 0))
    full_b = pl.BlockSpec((1, D), lambda i: (0, 0))
    return pl.pallas_call(
        _mlp_body,
        grid=grid,
        in_specs=[p_spec, row_spec, full_w, full_b, full_w, full_b],
        out_specs=row_spec,
        out_shape=jax.ShapeDtypeStruct((N, D), jnp.float32),
    )(p, h, W1, b1.reshape(1, D), W2, b2.reshape(1, D))


def kernel(x, edge_index, W1_0, b1_0, W2_0, b2_0, W1_1, b1_1, W2_1, b2_1):
    src = edge_index[0]
    dst = edge_index[1].reshape(NW, NCHUNK, K)
    p = _sc_aggregate(x, src, dst)
    h1 = _tc_mlp(p, x, W1_0, b1_0, W2_0, b2_0)
    p2 = _sc_aggregate(h1, src, dst)
    return _tc_mlp(p2, h1, W1_1, b1_1, W2_1, b2_1)
